# baseline (speedup 1.0000x reference)
import functools

import jax
import jax.numpy as jnp
from jax import lax
from jax.experimental import pallas as pl
from jax.experimental.pallas import tpu as pltpu
from jax.experimental.pallas import tpu_sc as plsc


_ROWS_PER_WORKER = 4000
_N_WORKERS = 25


def _sc_copy(x_hbm, o_hbm, sem):
    wid = lax.axis_index("s") * 2 + lax.axis_index("c")

    @pl.when(wid < _N_WORKERS)
    def _():
        base = wid * _ROWS_PER_WORKER
        pltpu.async_copy(
            x_hbm.at[pl.ds(base, _ROWS_PER_WORKER)],
            o_hbm.at[pl.ds(base, _ROWS_PER_WORKER)],
            sem,
        ).wait()


def kernel(x):
    mesh = plsc.VectorSubcoreMesh(core_axis_name="c", subcore_axis_name="s")
    run = functools.partial(
        pl.kernel,
        _sc_copy,
        mesh=mesh,
        out_type=jax.ShapeDtypeStruct(x.shape, x.dtype),
        scratch_types=[pltpu.SemaphoreType.DMA],
    )()
    return run(x)


# speedup vs baseline: 28.1872x; 28.1872x over previous
import functools

import jax
import jax.numpy as jnp
from jax import lax
from jax.experimental import pallas as pl
from jax.experimental.pallas import tpu as pltpu
from jax.experimental.pallas import tpu_sc as plsc


_ROWS_PER_WORKER = 4000
_N_WORKERS = 25
_CHUNK = 400
_N_CHUNKS = _ROWS_PER_WORKER // _CHUNK


def _sc_copy(x_hbm, o_hbm, buf0, buf1, sem_in, sem_out):
    wid = lax.axis_index("s") * 2 + lax.axis_index("c")

    @pl.when(wid < _N_WORKERS)
    def _():
        base = wid * _ROWS_PER_WORKER
        bufs = (buf0, buf1)
        ins = [
            pltpu.make_async_copy(
                x_hbm.at[pl.ds(base + j * _CHUNK, _CHUNK)],
                bufs[j % 2],
                sem_in.at[j % 2],
            )
            for j in range(_N_CHUNKS)
        ]
        outs = [
            pltpu.make_async_copy(
                bufs[j % 2],
                o_hbm.at[pl.ds(base + j * _CHUNK, _CHUNK)],
                sem_out.at[j % 2],
            )
            for j in range(_N_CHUNKS)
        ]
        ins[0].start()
        for j in range(_N_CHUNKS):
            if j + 1 < _N_CHUNKS:
                if j >= 1:
                    outs[j - 1].wait()
                ins[j + 1].start()
            ins[j].wait()
            outs[j].start()
        if _N_CHUNKS >= 2:
            outs[_N_CHUNKS - 2].wait()
        outs[_N_CHUNKS - 1].wait()


def kernel(x):
    mesh = plsc.VectorSubcoreMesh(core_axis_name="c", subcore_axis_name="s")
    run = functools.partial(
        pl.kernel,
        _sc_copy,
        mesh=mesh,
        out_type=jax.ShapeDtypeStruct(x.shape, x.dtype),
        scratch_types=[
            pltpu.VMEM((_CHUNK, 128), jnp.float32),
            pltpu.VMEM((_CHUNK, 128), jnp.float32),
            pltpu.SemaphoreType.DMA((2,)),
            pltpu.SemaphoreType.DMA((2,)),
        ],
    )()
    return run(x)


# speedup vs baseline: 49.7713x; 1.7657x over previous
import jax
import jax.numpy as jnp
from jax.experimental import pallas as pl
from jax.experimental.pallas import tpu as pltpu


_BLOCK = 25000


def _copy_kernel(x_ref, o_ref):
    o_ref[...] = x_ref[...]


def kernel(x):
    rows, feat = x.shape
    return pl.pallas_call(
        _copy_kernel,
        grid=(pl.cdiv(rows, _BLOCK),),
        in_specs=[pl.BlockSpec((_BLOCK, feat), lambda i: (i, 0))],
        out_specs=pl.BlockSpec((_BLOCK, feat), lambda i: (i, 0)),
        out_shape=jax.ShapeDtypeStruct(x.shape, x.dtype),
    )(x)


# speedup vs baseline: 51.2360x; 1.0294x over previous
import jax
import jax.numpy as jnp
from jax.experimental import pallas as pl
from jax.experimental.pallas import tpu as pltpu


_BLOCK = 30000


def _copy_kernel(x_ref, o_ref):
    o_ref[...] = x_ref[...]


def kernel(x):
    rows, feat = x.shape
    return pl.pallas_call(
        _copy_kernel,
        grid=(pl.cdiv(rows, _BLOCK),),
        in_specs=[pl.BlockSpec((_BLOCK, feat), lambda i: (i, 0))],
        out_specs=pl.BlockSpec((_BLOCK, feat), lambda i: (i, 0)),
        out_shape=jax.ShapeDtypeStruct(x.shape, x.dtype),
    )(x)
